# baseline (speedup 1.0000x reference)
import functools

import jax
import jax.numpy as jnp
from jax.experimental import pallas as pl
from jax.experimental.pallas import tpu as pltpu


@functools.lru_cache(maxsize=None)
def _make_kernel(n_lookups, d):
    def body(idx_ref, row_in, row_out):
        row_out[...] = row_in[...]

    grid_spec = pltpu.PrefetchScalarGridSpec(
        num_scalar_prefetch=1,
        grid=(n_lookups,),
        in_specs=[
            pl.BlockSpec((1, 1, d), lambda i, idx_ref: (idx_ref[i], 0, 0))
        ],
        out_specs=pl.BlockSpec((1, 1, d), lambda i, idx_ref: (i, 0, 0)),
    )
    return pl.pallas_call(
        body,
        grid_spec=grid_spec,
        out_shape=jax.ShapeDtypeStruct((n_lookups, 1, d), jnp.float32),
    )


def kernel(indices, table):
    b, t = indices.shape
    n_lookups = b * t
    d = table.shape[1]
    kern = _make_kernel(n_lookups, d)
    idx = indices.reshape(n_lookups).astype(jnp.int32)
    out = kern(idx, table.reshape(table.shape[0], 1, d))
    return out.reshape(b, t, d)


# speedup vs baseline: 21.7392x; 21.7392x over previous
import functools

import jax
import jax.numpy as jnp
from jax import lax
from jax.experimental import pallas as pl
from jax.experimental.pallas import tpu as pltpu
from jax.experimental.pallas import tpu_sc as plsc

_K = 2
_N = 6
_IPAD = 8


@functools.lru_cache(maxsize=None)
def _make_kernel(n_lookups, d):
    info = plsc.get_sparse_core_info()
    nw = info.num_cores * info.num_subcores
    b_per_w = n_lookups // nw
    m = b_per_w // _K
    half = _N // 2

    mesh = plsc.VectorSubcoreMesh(core_axis_name="c", subcore_axis_name="s")

    @functools.partial(
        pl.kernel,
        mesh=mesh,
        out_type=jax.ShapeDtypeStruct((n_lookups, d), jnp.float32),
        scratch_types=(
            [pltpu.VMEM((m, _IPAD), jnp.int32)]
            + [pltpu.VMEM((_K, d), jnp.float32) for _ in range(_N)]
            + [pltpu.SemaphoreType.DMA for _ in range(2 * _N)]
        ),
    )
    def kern(idx_hbm, table_hbm, out_hbm, idx_v, *rest):
        bufs = rest[:_N]
        gsems = rest[_N:2 * _N]
        ssems = rest[2 * _N:]

        wid = lax.axis_index("s") * info.num_cores + lax.axis_index("c")
        base = wid * b_per_w
        pltpu.sync_copy(idx_hbm.at[wid], idx_v)

        def gather(c, t):
            pltpu.async_copy(
                table_hbm.at[idx_v.at[c, pl.ds(0, _K)]], bufs[t], gsems[t]
            )

        def gwait(t):
            pltpu.make_async_copy(
                table_hbm.at[pl.ds(0, _K)], bufs[t], gsems[t]
            ).wait()

        def scatter(c, t):
            pltpu.async_copy(
                bufs[t], out_hbm.at[pl.ds(base + c * _K, _K)], ssems[t]
            )

        def swait(t):
            pltpu.make_async_copy(
                bufs[t], out_hbm.at[pl.ds(0, _K)], ssems[t]
            ).wait()


        for c in range(half):
            gather(c, c % _N)
        for c in range(half):
            gwait(c % _N)
            scatter(c, c % _N)
            gather(c + half, (c + half) % _N)

        for c in range(half, _N):
            gwait(c % _N)
            scatter(c, c % _N)
            swait((c + half) % _N)
            gather(c + half, (c + half) % _N)

        n_body = (m - _N - half) // _N
        n_steady = n_body * _N
        assert m == _N + n_steady + half + (m - _N - n_steady - half)
        rem = m - _N - n_steady - half

        def body(i, carry):
            c = _N + _N * i
            for t in range(_N):
                bt = t
                gwait(bt)
                scatter(c + t, bt)
                t2 = (t + half) % _N
                swait(t2)
                gather(c + t + half, t2)
            return carry

        lax.fori_loop(0, n_body, body, 0)

        for j in range(rem):
            c = _N + n_steady + j
            t = c % _N
            gwait(t)
            scatter(c, t)
            t2 = (c + half) % _N
            swait(t2)
            gather(c + half, t2)

        for j in range(half):
            c = m - half + j
            t = c % _N
            gwait(t)
            scatter(c, t)
        for t in range(_N):
            swait(t)

    return kern, nw, m


def kernel(indices, table):
    b, t = indices.shape
    n_lookups = b * t
    kern, nw, m = _make_kernel(n_lookups, table.shape[1])
    idx = indices.reshape(nw, m, _K).astype(jnp.int32)
    idx = jnp.pad(idx, ((0, 0), (0, 0), (0, _IPAD - _K)))
    out = kern(idx, table)
    return out.reshape(b, t, table.shape[1])


# speedup vs baseline: 21.8494x; 1.0051x over previous
import functools

import jax
import jax.numpy as jnp
from jax import lax
from jax.experimental import pallas as pl
from jax.experimental.pallas import tpu as pltpu
from jax.experimental.pallas import tpu_sc as plsc

_K = 4
_IPAD = 4


@functools.lru_cache(maxsize=None)
def _make_kernel(n_lookups, d):
    info = plsc.get_sparse_core_info()
    nw = info.num_cores * info.num_subcores
    b_per_w = n_lookups // nw
    n_chunks = b_per_w // _K
    n_body = (n_chunks - 4) // 3
    assert n_chunks == 1 + 3 * n_body + 3

    mesh = plsc.VectorSubcoreMesh(core_axis_name="c", subcore_axis_name="s")

    @functools.partial(
        pl.kernel,
        mesh=mesh,
        out_type=jax.ShapeDtypeStruct((n_lookups, d), jnp.float32),
        scratch_types=[
            pltpu.VMEM((n_chunks, _IPAD), jnp.int32),
            pltpu.VMEM((_K, d), jnp.float32),
            pltpu.VMEM((_K, d), jnp.float32),
            pltpu.VMEM((_K, d), jnp.float32),
            pltpu.SemaphoreType.DMA,
            pltpu.SemaphoreType.DMA,
            pltpu.SemaphoreType.DMA,
            pltpu.SemaphoreType.DMA,
            pltpu.SemaphoreType.DMA,
            pltpu.SemaphoreType.DMA,
        ],
    )
    def kern(idx_hbm, table_hbm, out_hbm, idx_v,
             buf_a, buf_b, buf_c, ga, gb, gc, sa, sb, sc):
        wid = lax.axis_index("s") * info.num_cores + lax.axis_index("c")
        base = wid * b_per_w
        pltpu.sync_copy(idx_hbm.at[wid], idx_v)

        bufs = (buf_a, buf_b, buf_c)
        gsems = (ga, gb, gc)
        ssems = (sa, sb, sc)

        def gather(c, t):
            pltpu.async_copy(
                table_hbm.at[idx_v.at[c, pl.ds(0, _K)]], bufs[t], gsems[t]
            )

        def gwait(t):
            pltpu.make_async_copy(
                table_hbm.at[pl.ds(0, _K)], bufs[t], gsems[t]
            ).wait()

        def scatter(c, t):
            pltpu.async_copy(
                bufs[t], out_hbm.at[pl.ds(base + c * _K, _K)], ssems[t]
            )

        def swait(t):
            pltpu.make_async_copy(
                bufs[t], out_hbm.at[pl.ds(0, _K)], ssems[t]
            ).wait()

        gather(0, 0)
        gather(1, 1)
        gwait(0)
        scatter(0, 0)
        gather(2, 2)

        def body(i, carry):
            c = 3 * i + 1
            for t in range(3):
                bt = (1 + t) % 3
                nxt = t % 3
                gwait(bt)
                scatter(c + t, bt)
                swait(nxt)
                gather(c + t + 2, nxt)
            return carry

        lax.fori_loop(0, n_body, body, 0)

        cl = n_chunks - 3
        gwait(1)
        scatter(cl, 1)
        swait(0)
        gather(cl + 2, 0)
        gwait(2)
        scatter(cl + 1, 2)
        gwait(0)
        scatter(cl + 2, 0)
        swait(1)
        swait(2)
        swait(0)

    return kern, nw, n_chunks


def kernel(indices, table):
    b, t = indices.shape
    n_lookups = b * t
    kern, nw, n_chunks = _make_kernel(n_lookups, table.shape[1])
    idx = indices.reshape(nw, n_chunks, _K).astype(jnp.int32)
    idx = jnp.pad(idx, ((0, 0), (0, 0), (0, _IPAD - _K)))
    out = kern(idx, table)
    return out.reshape(b, t, table.shape[1])
